# baseline (speedup 1.0000x reference)
import jax
import jax.numpy as jnp
import numpy as np
from jax import lax
from jax.experimental import pallas as pl
from jax.experimental.pallas import tpu as pltpu, tpu_sc as plsc

BATCH = 16384
NUM_FIELDS = 26
NUM_HEADS = 2
TABLE_ROWS = 2600000
S4 = 655360
WQ = (0, 647168, 1302528, 1949696)
TQ = (655360, 1302528, 1957888)
ROWS_PER_SAMPLE = 52

NC, NS = 2, 16
NW = NC * NS
NB = BATCH // NW
CS = 32
NCH = NB // CS
CR = CS * ROWS_PER_SAMPLE
NIDX = CR // 128
XW = NB * NUM_FIELDS

TC_BC = 8192
TC_G = S4 // TC_BC


def _static_tables():
    k = np.arange(CR)
    bloc, j = k // ROWS_PER_SAMPLE, k % ROWS_PER_SAMPLE
    f, e = j % NUM_FIELDS, j // NUM_FIELDS
    mrel = (bloc * NUM_FIELDS + f).astype(np.int32)
    vadd = (8 * 100000 * f + e).astype(np.int32)
    vthr = (8 * TQ[0] + e).astype(np.int32)
    return jnp.asarray(mrel), jnp.asarray(vadd), jnp.asarray(vthr)


_DQ = tuple(2 * q - 8 * WQ[q] for q in range(4))
_DT1 = 8 * (TQ[1] - TQ[0])
_DT2 = 8 * (TQ[2] - TQ[1])


def _tr_body(a_ref, b_ref, c_ref, d_ref, eye_ref, out_ref):
    z = jnp.concatenate(
        [a_ref[...], b_ref[...], c_ref[...], d_ref[...]], axis=0)
    out_ref[...] = lax.dot_general(
        z, eye_ref[...], (((0,), (0,)), ((), ())),
        precision=lax.Precision.HIGHEST,
        preferred_element_type=jnp.float32)


def _sc_body(x_hbm, mrel_hbm, vadd_hbm, vthr_hbm, table4_hbm, out_hbm,
             x_v, mrel_v, vadd_v, vthr_v, idx0, idx1, dest0, dest1,
             g0, g1, s0, s1):
    wid = lax.axis_index("s") * NC + lax.axis_index("c")
    pltpu.sync_copy(x_hbm.at[pl.ds(wid * XW, XW)], x_v)
    pltpu.sync_copy(mrel_hbm, mrel_v)
    pltpu.sync_copy(vadd_hbm, vadd_v)
    pltpu.sync_copy(vthr_hbm, vthr_v)
    row_base = wid * (NB * ROWS_PER_SAMPLE)

    def compute_idx(c, idx_v):
        base_m = c * (CS * NUM_FIELDS)
        for t in range(CR // 16):
            vm = mrel_v[pl.ds(t * 16, 16)]
            va = vadd_v[pl.ds(t * 16, 16)]
            vt1 = vthr_v[pl.ds(t * 16, 16)]
            vx = plsc.load_gather(x_v, [vm + base_m])
            u = (vx << 3) + va
            idx = u + _DQ[0]
            idx = jnp.where(u >= vt1, idx + (_DQ[1] - _DQ[0]), idx)
            idx = jnp.where(u >= vt1 + _DT1, idx + (_DQ[2] - _DQ[1]), idx)
            idx = jnp.where(u >= vt1 + (_DT1 + _DT2),
                            idx + (_DQ[3] - _DQ[2]), idx)
            idx_v[t // 8, pl.ds((t % 8) * 16, 16)] = idx

    def fire_gathers(idx_v, dest_v, sem):
        for jj in range(NIDX):
            pltpu.async_copy(table4_hbm.at[idx_v.at[jj]],
                             dest_v.at[pl.ds(jj * 128, 128)], sem)

    def drain(dest_v, sem):
        pltpu.make_async_copy(out_hbm.at[pl.ds(0, CR)], dest_v, sem).wait()

    def fire_out(c, dest_v, sem):
        pltpu.async_copy(dest_v, out_hbm.at[pl.ds(row_base + c * CR, CR)], sem)

    compute_idx(0, idx0)
    fire_gathers(idx0, dest0, g0)
    compute_idx(1, idx1)
    fire_gathers(idx1, dest1, g1)

    def body(c2, _):
        ca = 2 * c2
        drain(dest0, g0)
        fire_out(ca, dest0, s0)
        drain(dest1, g1)
        fire_out(ca + 1, dest1, s1)
        compute_idx(ca + 2, idx0)
        drain(dest0, s0)
        fire_gathers(idx0, dest0, g0)
        compute_idx(ca + 3, idx1)
        drain(dest1, s1)
        fire_gathers(idx1, dest1, g1)
        return ()

    lax.fori_loop(0, NCH // 2 - 1, body, (), unroll=False)

    drain(dest0, g0)
    fire_out(NCH - 2, dest0, s0)
    drain(dest1, g1)
    fire_out(NCH - 1, dest1, s1)
    drain(dest0, s0)
    drain(dest1, s1)


def _transpose_head(tT, eye, h):
    wblk = [w // TC_BC for w in WQ]
    return pl.pallas_call(
        _tr_body,
        grid=(TC_G,),
        in_specs=[
            pl.BlockSpec((32, TC_BC), lambda i, w=w: (h, i + w))
            for w in wblk
        ] + [pl.BlockSpec((128, 128), lambda i: (0, 0))],
        out_specs=pl.BlockSpec((TC_BC, 128), lambda i: (i, 0)),
        out_shape=jax.ShapeDtypeStruct((S4, 128), jnp.float32),
    )(tT, tT, tT, tT, eye)


def _gather_head(x_flat, tables, table4):
    return pl.kernel(
        _sc_body,
        out_type=jax.ShapeDtypeStruct((BATCH * ROWS_PER_SAMPLE, 16),
                                      jnp.float32),
        mesh=plsc.VectorSubcoreMesh(core_axis_name="c", subcore_axis_name="s"),
        scratch_types=[
            pltpu.VMEM((XW,), jnp.int32),
            pltpu.VMEM((CR,), jnp.int32),
            pltpu.VMEM((CR,), jnp.int32),
            pltpu.VMEM((CR,), jnp.int32),
            pltpu.VMEM((NIDX, 128), jnp.int32),
            pltpu.VMEM((NIDX, 128), jnp.int32),
            pltpu.VMEM((CR, 16), jnp.float32),
            pltpu.VMEM((CR, 16), jnp.float32),
            pltpu.SemaphoreType.DMA,
            pltpu.SemaphoreType.DMA,
            pltpu.SemaphoreType.DMA,
            pltpu.SemaphoreType.DMA,
        ],
        compiler_params=pltpu.CompilerParams(needs_layout_passes=False,
                                             use_tc_tiling_on_sc=False),
    )(x_flat, *tables, table4)


def kernel(X, table):
    tT = table.T
    eye = jnp.eye(128, dtype=jnp.float32)
    tables = _static_tables()
    x_flat = X.reshape(-1)
    outs = []
    for h in range(NUM_HEADS):
        t128 = _transpose_head(tT, eye, h)
        o = _gather_head(x_flat, tables, t128.reshape(-1, 16))
        outs.append(o.reshape(BATCH, ROWS_PER_SAMPLE * 16))
    return jnp.stack(outs, axis=1)


# speedup vs baseline: 1.1867x; 1.1867x over previous
import jax
import jax.numpy as jnp
import numpy as np
from jax import lax
from jax.experimental import pallas as pl
from jax.experimental.pallas import tpu as pltpu, tpu_sc as plsc

BATCH = 16384
NUM_FIELDS = 26
NUM_HEADS = 2
TABLE_ROWS = 2600000
S4 = 655360
WQ = (0, 647168, 1302528, 1949696)
TQ = (655360, 1302528, 1957888)
ROWS_PER_SAMPLE = 52

NC, NS = 2, 16
NW = NC * NS
NB = BATCH // NW
CS = 32
NCH = NB // CS
CR = CS * ROWS_PER_SAMPLE
NIDX = CR // 128
XW = NB * NUM_FIELDS

TC_BC = 8192
TC_G = S4 // TC_BC


def _static_tables():
    k = np.arange(CR)
    bloc, j = k // ROWS_PER_SAMPLE, k % ROWS_PER_SAMPLE
    f, e = j % NUM_FIELDS, j // NUM_FIELDS
    mrel = (bloc * NUM_FIELDS + f).astype(np.int32)
    vadd = (8 * 100000 * f + e).astype(np.int32)
    vthr = (8 * TQ[0] + e).astype(np.int32)
    return jnp.asarray(mrel), jnp.asarray(vadd), jnp.asarray(vthr)


_DQ = tuple(2 * q - 8 * WQ[q] for q in range(4))
_DT1 = 8 * (TQ[1] - TQ[0])
_DT2 = 8 * (TQ[2] - TQ[1])


def _tr_body(a_ref, b_ref, c_ref, d_ref, eye_ref, out_ref):
    z = jnp.concatenate(
        [a_ref[...], b_ref[...], c_ref[...], d_ref[...]], axis=0)
    out_ref[...] = lax.dot_general(
        z, eye_ref[...], (((0,), (0,)), ((), ())),
        preferred_element_type=jnp.float32)


def _sc_body(x_hbm, mrel_hbm, vadd_hbm, vthr_hbm, table4_hbm, out_hbm,
             x_v, mrel_v, vadd_v, vthr_v, idx0, idx1, dest0, dest1,
             g0, g1, s0, s1):
    wid = lax.axis_index("s") * NC + lax.axis_index("c")
    pltpu.sync_copy(x_hbm.at[pl.ds(wid * XW, XW)], x_v)
    pltpu.sync_copy(mrel_hbm, mrel_v)
    pltpu.sync_copy(vadd_hbm, vadd_v)
    pltpu.sync_copy(vthr_hbm, vthr_v)
    row_base = wid * (NB * ROWS_PER_SAMPLE)

    def compute_idx(c, idx_v):
        base_m = c * (CS * NUM_FIELDS)
        for t in range(CR // 16):
            vm = mrel_v[pl.ds(t * 16, 16)]
            va = vadd_v[pl.ds(t * 16, 16)]
            vt1 = vthr_v[pl.ds(t * 16, 16)]
            vx = plsc.load_gather(x_v, [vm + base_m])
            u = (vx << 3) + va
            idx = u + _DQ[0]
            idx = jnp.where(u >= vt1, idx + (_DQ[1] - _DQ[0]), idx)
            idx = jnp.where(u >= vt1 + _DT1, idx + (_DQ[2] - _DQ[1]), idx)
            idx = jnp.where(u >= vt1 + (_DT1 + _DT2),
                            idx + (_DQ[3] - _DQ[2]), idx)
            idx_v[t // 8, pl.ds((t % 8) * 16, 16)] = idx

    def fire_gathers(idx_v, dest_v, sem):
        for jj in range(NIDX):
            pltpu.async_copy(table4_hbm.at[idx_v.at[jj]],
                             dest_v.at[pl.ds(jj * 128, 128)], sem)

    def drain(dest_v, sem):
        pltpu.make_async_copy(out_hbm.at[pl.ds(0, CR)], dest_v, sem).wait()

    def fire_out(c, dest_v, sem):
        pltpu.async_copy(dest_v, out_hbm.at[pl.ds(row_base + c * CR, CR)], sem)

    compute_idx(0, idx0)
    fire_gathers(idx0, dest0, g0)
    compute_idx(1, idx1)
    fire_gathers(idx1, dest1, g1)

    def body(c2, _):
        ca = 2 * c2
        drain(dest0, g0)
        fire_out(ca, dest0, s0)
        drain(dest1, g1)
        fire_out(ca + 1, dest1, s1)
        compute_idx(ca + 2, idx0)
        drain(dest0, s0)
        fire_gathers(idx0, dest0, g0)
        compute_idx(ca + 3, idx1)
        drain(dest1, s1)
        fire_gathers(idx1, dest1, g1)
        return ()

    lax.fori_loop(0, NCH // 2 - 1, body, (), unroll=False)

    drain(dest0, g0)
    fire_out(NCH - 2, dest0, s0)
    drain(dest1, g1)
    fire_out(NCH - 1, dest1, s1)
    drain(dest0, s0)
    drain(dest1, s1)


def _transpose_head(tT, eye, h):
    wblk = [w // TC_BC for w in WQ]
    return pl.pallas_call(
        _tr_body,
        grid=(TC_G,),
        in_specs=[
            pl.BlockSpec((32, TC_BC), lambda i, w=w: (h, i + w))
            for w in wblk
        ] + [pl.BlockSpec((128, 128), lambda i: (0, 0))],
        out_specs=pl.BlockSpec((TC_BC, 128), lambda i: (i, 0)),
        out_shape=jax.ShapeDtypeStruct((S4, 128), jnp.float32),
    )(tT, tT, tT, tT, eye)


def _gather_head(x_flat, tables, table4):
    return pl.kernel(
        _sc_body,
        out_type=jax.ShapeDtypeStruct((BATCH * ROWS_PER_SAMPLE, 16),
                                      jnp.float32),
        mesh=plsc.VectorSubcoreMesh(core_axis_name="c", subcore_axis_name="s"),
        scratch_types=[
            pltpu.VMEM((XW,), jnp.int32),
            pltpu.VMEM((CR,), jnp.int32),
            pltpu.VMEM((CR,), jnp.int32),
            pltpu.VMEM((CR,), jnp.int32),
            pltpu.VMEM((NIDX, 128), jnp.int32),
            pltpu.VMEM((NIDX, 128), jnp.int32),
            pltpu.VMEM((CR, 16), jnp.float32),
            pltpu.VMEM((CR, 16), jnp.float32),
            pltpu.SemaphoreType.DMA,
            pltpu.SemaphoreType.DMA,
            pltpu.SemaphoreType.DMA,
            pltpu.SemaphoreType.DMA,
        ],
        compiler_params=pltpu.CompilerParams(needs_layout_passes=False,
                                             use_tc_tiling_on_sc=False),
    )(x_flat, *tables, table4)


def kernel(X, table):
    tT = table.T
    eye = jnp.eye(128, dtype=jnp.float32)
    tables = _static_tables()
    x_flat = X.reshape(-1)
    outs = []
    for h in range(NUM_HEADS):
        t128 = _transpose_head(tT, eye, h)
        o = _gather_head(x_flat, tables, t128.reshape(-1, 16))
        outs.append(o.reshape(BATCH, ROWS_PER_SAMPLE * 16))
    return jnp.stack(outs, axis=1)
